# baseline (speedup 1.0000x reference)
import functools

import jax
import jax.numpy as jnp
from jax import lax
from jax.experimental import pallas as pl
from jax.experimental.pallas import tpu as pltpu
from jax.experimental.pallas import tpu_sc as plsc

NC = 2
NS = 16
CHUNK = 128
EXPORT_CHUNK = 128
LANES = 16


def _sc_edge_kernel(d_feat, cpw, npad, mode, spmem_table=False):
    rows_pt = npad // NS
    cpw_eff = 2 * cpw if mode == "split" else cpw
    mesh = plsc.VectorSubcoreMesh(core_axis_name="c", subcore_axis_name="s")

    scratch = [
        pltpu.VMEM((cpw_eff, CHUNK), jnp.int32),
        pltpu.VMEM((CHUNK, d_feat), jnp.float32),
        pltpu.VMEM((EXPORT_CHUNK, d_feat), jnp.float32),
        pltpu.VMEM_SHARED((npad, d_feat), jnp.float32),
        pltpu.SemaphoreType.DMA,
    ]
    if mode != "deg":
        scratch = [pltpu.VMEM((cpw_eff, CHUNK), jnp.int32)] + scratch
        if spmem_table:
            scratch = scratch + [
                pltpu.VMEM_SHARED((npad, d_feat), jnp.float32)]

    def body(*refs):
        table_sh = None
        if mode == "deg":
            (dst_hbm, out_hbm, dst_v, rows_v, stage_v, acc_sh, sem) = refs
        elif spmem_table:
            (table_hbm, src_hbm, dst_hbm, out_hbm,
             src_v, dst_v, rows_v, stage_v, acc_sh, sem, table_sh) = refs
        else:
            (table_hbm, src_hbm, dst_hbm, out_hbm,
             src_v, dst_v, rows_v, stage_v, acc_sh, sem) = refs
        cid = lax.axis_index("c")
        sid = lax.axis_index("s")

        zero16 = jnp.zeros((LANES,), jnp.float32)

        def zero_stage(i, carry):
            for j in range(d_feat // LANES):
                stage_v[i, pl.ds(j * LANES, LANES)] = zero16
            return carry
        lax.fori_loop(0, EXPORT_CHUNK, zero_stage, 0)

        if mode == "deg":
            one16 = jnp.full((LANES,), 1.0, jnp.float32)

            def fill_ones(i, carry):
                for j in range(d_feat // LANES):
                    rows_v[i, pl.ds(j * LANES, LANES)] = one16
                return carry
            lax.fori_loop(0, CHUNK, fill_ones, 0)

        def zero_acc(i, carry):
            r0 = sid * rows_pt + i * EXPORT_CHUNK
            pltpu.sync_copy(stage_v, acc_sh.at[pl.ds(r0, EXPORT_CHUNK)])
            return carry
        lax.fori_loop(0, rows_pt // EXPORT_CHUNK, zero_acc, 0)

        if table_sh is not None:
            def load_table(i, carry):
                r0 = sid * rows_pt + i * EXPORT_CHUNK
                h0 = cid * npad + r0 if mode == "split" else r0
                pltpu.sync_copy(table_hbm.at[pl.ds(h0, EXPORT_CHUNK)], stage_v)
                pltpu.sync_copy(stage_v, table_sh.at[pl.ds(r0, EXPORT_CHUNK)])
                return carry
            lax.fori_loop(0, rows_pt // EXPORT_CHUNK, load_table, 0)

        src_plane = 0 if spmem_table else cid
        if mode == "split":
            pltpu.sync_copy(dst_hbm.at[sid], dst_v)
            pltpu.sync_copy(src_hbm.at[src_plane, sid], src_v)
        else:
            col = cid * cpw
            pltpu.sync_copy(dst_hbm.at[sid, pl.ds(col, cpw)], dst_v)
            if mode == "partial":
                pltpu.sync_copy(src_hbm.at[0, sid, pl.ds(col, cpw)], src_v)

        plsc.subcore_barrier()

        table = table_sh if table_sh is not None else (
            None if mode == "deg" else table_hbm)

        def edge_body(j, carry):
            if table is not None:
                pltpu.async_copy(table.at[src_v.at[j]], rows_v, sem).wait()
            pltpu.sync_copy(rows_v, acc_sh.at[dst_v.at[j]], add=True)
            return carry
        lax.fori_loop(0, cpw_eff, edge_body, 0)

        plsc.subcore_barrier()

        def export(i, carry):
            r0 = sid * rows_pt + i * EXPORT_CHUNK
            pltpu.sync_copy(acc_sh.at[pl.ds(r0, EXPORT_CHUNK)], stage_v)
            pltpu.sync_copy(stage_v, out_hbm.at[cid, pl.ds(r0, EXPORT_CHUNK)])
            return carry
        lax.fori_loop(0, rows_pt // EXPORT_CHUNK, export, 0)

    return functools.partial(
        pl.kernel,
        out_type=jax.ShapeDtypeStruct((NC, npad, d_feat), jnp.float32),
        mesh=mesh,
        scratch_types=scratch,
        compiler_params=pltpu.CompilerParams(use_tc_tiling_on_sc=False),
    )(body)


BLK = 256
_DOT = dict(preferred_element_type=jnp.float32, precision=lax.Precision.HIGHEST)


def _tc_scale_matmul(x_ref, w1_ref, d0_ref, d1_ref, out_ref):
    dis = lax.rsqrt(d0_ref[:, :1] + d1_ref[:, :1] + 1.0)
    h = jnp.dot(x_ref[:, :], w1_ref[:, :], **_DOT) * dis
    half = h.shape[1] // 2
    out_ref[0, :, :] = h[:, :half]
    out_ref[1, :, :] = h[:, half:]


def _tc_mid(ha_ref, hb_ref, ga_ref, gb_ref,
            d0_ref, d1_ref, b1_ref, w2_ref, wo_ref, out_ref):
    dis = lax.rsqrt(d0_ref[:, :1] + d1_ref[:, :1] + 1.0)
    agg = jnp.concatenate([ga_ref[:, :] + ha_ref[:, :],
                           gb_ref[:, :] + hb_ref[:, :]], axis=1)
    z = jnp.maximum(dis * agg + b1_ref[:, :], 0.0)
    h2 = jnp.dot(jnp.dot(z, w2_ref[:, :], **_DOT), wo_ref[:, :], **_DOT)
    out_ref[:, :] = h2 * dis


def _tc_head(q0_ref, q1_ref, h2p_ref, d0_ref, d1_ref, b2_ref, wo_ref, bo_ref,
             out_ref, *, cpad, nclass):
    dis = lax.rsqrt(d0_ref[:, :1] + d1_ref[:, :1] + 1.0)
    bias = jnp.dot(b2_ref[:, :], wo_ref[:, :], **_DOT) + bo_ref[:, :]
    u = dis * (q0_ref[:, :] + q1_ref[:, :] + h2p_ref[:, :]) + bias
    col = lax.broadcasted_iota(jnp.int32, (BLK, cpad), 1)
    valid = col < nclass
    um = jnp.where(valid, u, jnp.float32(-1e30))
    m = jnp.max(um, axis=1, keepdims=True)
    e = jnp.where(valid, jnp.exp(um - m), 0.0)
    s = jnp.sum(e, axis=1, keepdims=True)
    out_ref[:, :] = u - m - jnp.log(s)


def _row_spec(width):
    return pl.BlockSpec((BLK, width), lambda i: (i, 0))


def _full_spec(shape):
    return pl.BlockSpec(shape, lambda i: (0,) * len(shape))


def kernel(x, edge_index, W1, b1, W2, b2, Wo, bo):
    n_nodes, d_feat = x.shape
    n_hid = W1.shape[1]
    half = n_hid // 2
    n_class = Wo.shape[1]
    cpad = ((n_class + LANES - 1) // LANES) * LANES
    align = NS * EXPORT_CHUNK
    npad = ((n_nodes + 1 + align - 1) // align) * align
    n_edges = edge_index.shape[1]
    cpw = (n_edges + NC * NS * CHUNK - 1) // (NC * NS * CHUNK)
    cpw = ((cpw + 3) // 4) * 4
    e_pad = NC * NS * CHUNK * cpw

    src = edge_index[0].astype(jnp.int32)
    dst = edge_index[1].astype(jnp.int32)
    fill = jnp.full((e_pad - n_edges,), n_nodes, jnp.int32)
    src_flat = jnp.concatenate([src, fill]).reshape(NS, NC * cpw, CHUNK)
    dst_slab = jnp.concatenate([dst, fill]).reshape(NS, NC * cpw, CHUNK)
    src_slab = jnp.stack([src_flat, src_flat + npad])

    x_pad = jnp.pad(x, ((0, npad - n_nodes), (0, 0)))
    wo_pad = jnp.pad(Wo, ((0, 0), (0, cpad - n_class)))
    bo_pad = jnp.pad(bo, (0, cpad - n_class)).reshape(1, cpad)
    b1_2d = b1.reshape(1, n_hid)
    b2_2d = b2.reshape(1, n_hid)

    grid = (npad // BLK,)

    degp = _sc_edge_kernel(LANES, cpw, npad, "deg")(dst_slab)
    d0, d1 = degp[0], degp[1]

    hs = pl.pallas_call(
        _tc_scale_matmul,
        grid=grid,
        in_specs=[_row_spec(d_feat), _full_spec((d_feat, n_hid)),
                  _row_spec(LANES), _row_spec(LANES)],
        out_specs=pl.BlockSpec((NC, BLK, half), lambda i: (0, i, 0)),
        out_shape=jax.ShapeDtypeStruct((NC, npad, half), jnp.float32),
    )(x_pad, W1, d0, d1)

    table = hs.reshape(NC * npad, half)
    agg = _sc_edge_kernel(half, cpw, npad, "split")(table, src_slab, dst_slab)

    h2p = pl.pallas_call(
        _tc_mid,
        grid=grid,
        in_specs=[_row_spec(half), _row_spec(half),
                  _row_spec(half), _row_spec(half),
                  _row_spec(LANES), _row_spec(LANES), _full_spec((1, n_hid)),
                  _full_spec((n_hid, n_hid)), _full_spec((n_hid, cpad))],
        out_specs=_row_spec(cpad),
        out_shape=jax.ShapeDtypeStruct((npad, cpad), jnp.float32),
    )(hs[0], hs[1], agg[0], agg[1], d0, d1, b1_2d, W2, wo_pad)

    agg2p = _sc_edge_kernel(cpad, cpw, npad, "partial")(h2p, src_slab, dst_slab)

    out = pl.pallas_call(
        functools.partial(_tc_head, cpad=cpad, nclass=n_class),
        grid=grid,
        in_specs=[_row_spec(cpad), _row_spec(cpad), _row_spec(cpad),
                  _row_spec(LANES), _row_spec(LANES), _full_spec((1, n_hid)),
                  _full_spec((n_hid, cpad)), _full_spec((1, cpad))],
        out_specs=_row_spec(cpad),
        out_shape=jax.ShapeDtypeStruct((npad, cpad), jnp.float32),
    )(agg2p[0], agg2p[1], h2p, d0, d1, b2_2d, wo_pad, bo_pad)

    return out[:n_nodes, :n_class]


# speedup vs baseline: 1.6374x; 1.6374x over previous
import functools

import jax
import jax.numpy as jnp
from jax import lax
from jax.experimental import pallas as pl
from jax.experimental.pallas import tpu as pltpu
from jax.experimental.pallas import tpu_sc as plsc

NC = 2
NS = 16
CHUNK = 128
EXPORT_CHUNK = 128
LANES = 16


def _sc_edge_kernel(d_feat, cpw, npad, mode, spmem_table=False):
    rows_pt = npad // NS
    cpw_eff = 2 * cpw if mode == "split" else cpw
    mesh = plsc.VectorSubcoreMesh(core_axis_name="c", subcore_axis_name="s")

    scratch = [
        pltpu.VMEM((cpw_eff, CHUNK), jnp.int32),
        pltpu.VMEM((CHUNK, d_feat), jnp.float32),
        pltpu.VMEM((EXPORT_CHUNK, d_feat), jnp.float32),
        pltpu.VMEM_SHARED((npad, d_feat), jnp.float32),
        pltpu.SemaphoreType.DMA,
    ]
    if mode != "deg":
        scratch = [pltpu.VMEM((cpw_eff, CHUNK), jnp.int32)] + scratch
        if spmem_table:
            scratch = scratch + [
                pltpu.VMEM_SHARED((npad, d_feat), jnp.float32)]

    def body(*refs):
        table_sh = None
        if mode == "deg":
            (dst_hbm, out_hbm, dst_v, rows_v, stage_v, acc_sh, sem) = refs
        elif spmem_table:
            (table_hbm, src_hbm, dst_hbm, out_hbm,
             src_v, dst_v, rows_v, stage_v, acc_sh, sem, table_sh) = refs
        else:
            (table_hbm, src_hbm, dst_hbm, out_hbm,
             src_v, dst_v, rows_v, stage_v, acc_sh, sem) = refs
        cid = lax.axis_index("c")
        sid = lax.axis_index("s")

        zero16 = jnp.zeros((LANES,), jnp.float32)

        def zero_stage(i, carry):
            for j in range(d_feat // LANES):
                stage_v[i, pl.ds(j * LANES, LANES)] = zero16
            return carry
        lax.fori_loop(0, EXPORT_CHUNK, zero_stage, 0)

        if mode == "deg":
            one16 = jnp.full((LANES,), 1.0, jnp.float32)

            def fill_ones(i, carry):
                for j in range(d_feat // LANES):
                    rows_v[i, pl.ds(j * LANES, LANES)] = one16
                return carry
            lax.fori_loop(0, CHUNK, fill_ones, 0)

        def zero_acc(i, carry):
            r0 = sid * rows_pt + i * EXPORT_CHUNK
            pltpu.sync_copy(stage_v, acc_sh.at[pl.ds(r0, EXPORT_CHUNK)])
            return carry
        lax.fori_loop(0, rows_pt // EXPORT_CHUNK, zero_acc, 0)

        if table_sh is not None:
            def load_table(i, carry):
                r0 = sid * rows_pt + i * EXPORT_CHUNK
                h0 = cid * npad + r0 if mode == "split" else r0
                pltpu.sync_copy(table_hbm.at[pl.ds(h0, EXPORT_CHUNK)], stage_v)
                pltpu.sync_copy(stage_v, table_sh.at[pl.ds(r0, EXPORT_CHUNK)])
                return carry
            lax.fori_loop(0, rows_pt // EXPORT_CHUNK, load_table, 0)

        src_plane = 0 if spmem_table else cid
        if mode == "split":
            pltpu.sync_copy(dst_hbm.at[sid], dst_v)
            pltpu.sync_copy(src_hbm.at[src_plane, sid], src_v)
        else:
            col = cid * cpw
            pltpu.sync_copy(dst_hbm.at[sid, pl.ds(col, cpw)], dst_v)
            if mode == "partial":
                pltpu.sync_copy(src_hbm.at[0, sid, pl.ds(col, cpw)], src_v)

        plsc.subcore_barrier()

        table = table_sh if table_sh is not None else (
            None if mode == "deg" else table_hbm)

        def edge_body(j, carry):
            if table is not None:
                pltpu.async_copy(table.at[src_v.at[j]], rows_v, sem).wait()
            pltpu.sync_copy(rows_v, acc_sh.at[dst_v.at[j]], add=True)
            return carry
        lax.fori_loop(0, cpw_eff, edge_body, 0)

        plsc.subcore_barrier()

        def export(i, carry):
            r0 = sid * rows_pt + i * EXPORT_CHUNK
            pltpu.sync_copy(acc_sh.at[pl.ds(r0, EXPORT_CHUNK)], stage_v)
            pltpu.sync_copy(stage_v, out_hbm.at[cid, pl.ds(r0, EXPORT_CHUNK)])
            return carry
        lax.fori_loop(0, rows_pt // EXPORT_CHUNK, export, 0)

    return functools.partial(
        pl.kernel,
        out_type=jax.ShapeDtypeStruct((NC, npad, d_feat), jnp.float32),
        mesh=mesh,
        scratch_types=scratch,
        compiler_params=pltpu.CompilerParams(use_tc_tiling_on_sc=False),
    )(body)


BLK = 256
_DOT = dict(preferred_element_type=jnp.float32, precision=lax.Precision.HIGHEST)


def _tc_scale_matmul(x_ref, w1_ref, d0_ref, d1_ref, out_ref):
    dis = lax.rsqrt(d0_ref[:, :1] + d1_ref[:, :1] + 1.0)
    h = jnp.dot(x_ref[:, :], w1_ref[:, :], **_DOT) * dis
    half = h.shape[1] // 2
    out_ref[0, :, :] = h[:, :half]
    out_ref[1, :, :] = h[:, half:]


def _tc_mid(ha_ref, hb_ref, ga_ref, gb_ref,
            d0_ref, d1_ref, b1_ref, w2_ref, wo_ref, out_ref):
    dis = lax.rsqrt(d0_ref[:, :1] + d1_ref[:, :1] + 1.0)
    agg = jnp.concatenate([ga_ref[:, :] + ha_ref[:, :],
                           gb_ref[:, :] + hb_ref[:, :]], axis=1)
    z = jnp.maximum(dis * agg + b1_ref[:, :], 0.0)
    h2 = jnp.dot(jnp.dot(z, w2_ref[:, :], **_DOT), wo_ref[:, :], **_DOT)
    out_ref[:, :] = h2 * dis


def _tc_head(q0_ref, q1_ref, h2p_ref, d0_ref, d1_ref, b2_ref, wo_ref, bo_ref,
             out_ref, *, cpad, nclass):
    dis = lax.rsqrt(d0_ref[:, :1] + d1_ref[:, :1] + 1.0)
    bias = jnp.dot(b2_ref[:, :], wo_ref[:, :], **_DOT) + bo_ref[:, :]
    u = dis * (q0_ref[:, :] + q1_ref[:, :] + h2p_ref[:, :]) + bias
    col = lax.broadcasted_iota(jnp.int32, (BLK, cpad), 1)
    valid = col < nclass
    um = jnp.where(valid, u, jnp.float32(-1e30))
    m = jnp.max(um, axis=1, keepdims=True)
    e = jnp.where(valid, jnp.exp(um - m), 0.0)
    s = jnp.sum(e, axis=1, keepdims=True)
    out_ref[:, :] = u - m - jnp.log(s)


def _row_spec(width):
    return pl.BlockSpec((BLK, width), lambda i: (i, 0))


def _full_spec(shape):
    return pl.BlockSpec(shape, lambda i: (0,) * len(shape))


def kernel(x, edge_index, W1, b1, W2, b2, Wo, bo):
    n_nodes, d_feat = x.shape
    n_hid = W1.shape[1]
    half = n_hid // 2
    n_class = Wo.shape[1]
    cpad = ((n_class + LANES - 1) // LANES) * LANES
    align = NS * EXPORT_CHUNK
    npad = ((n_nodes + 1 + align - 1) // align) * align
    n_edges = edge_index.shape[1]
    cpw = (n_edges + NC * NS * CHUNK - 1) // (NC * NS * CHUNK)
    e_pad = NC * NS * CHUNK * cpw

    src = edge_index[0].astype(jnp.int32)
    dst = edge_index[1].astype(jnp.int32)
    spare = npad - n_nodes
    fill = n_nodes + jnp.arange(e_pad - n_edges, dtype=jnp.int32) % spare
    src_flat = jnp.concatenate([src, fill]).reshape(NS, NC * cpw, CHUNK)
    dst_slab = jnp.concatenate([dst, fill]).reshape(NS, NC * cpw, CHUNK)
    src_slab = jnp.stack([src_flat, src_flat + npad])

    x_pad = jnp.pad(x, ((0, npad - n_nodes), (0, 0)))
    wo_pad = jnp.pad(Wo, ((0, 0), (0, cpad - n_class)))
    bo_pad = jnp.pad(bo, (0, cpad - n_class)).reshape(1, cpad)
    b1_2d = b1.reshape(1, n_hid)
    b2_2d = b2.reshape(1, n_hid)

    grid = (npad // BLK,)

    degp = _sc_edge_kernel(LANES, cpw, npad, "deg")(dst_slab)
    d0, d1 = degp[0], degp[1]

    hs = pl.pallas_call(
        _tc_scale_matmul,
        grid=grid,
        in_specs=[_row_spec(d_feat), _full_spec((d_feat, n_hid)),
                  _row_spec(LANES), _row_spec(LANES)],
        out_specs=pl.BlockSpec((NC, BLK, half), lambda i: (0, i, 0)),
        out_shape=jax.ShapeDtypeStruct((NC, npad, half), jnp.float32),
    )(x_pad, W1, d0, d1)

    table = hs.reshape(NC * npad, half)
    agg = _sc_edge_kernel(half, cpw, npad, "split")(table, src_slab, dst_slab)

    h2p = pl.pallas_call(
        _tc_mid,
        grid=grid,
        in_specs=[_row_spec(half), _row_spec(half),
                  _row_spec(half), _row_spec(half),
                  _row_spec(LANES), _row_spec(LANES), _full_spec((1, n_hid)),
                  _full_spec((n_hid, n_hid)), _full_spec((n_hid, cpad))],
        out_specs=_row_spec(cpad),
        out_shape=jax.ShapeDtypeStruct((npad, cpad), jnp.float32),
    )(hs[0], hs[1], agg[0], agg[1], d0, d1, b1_2d, W2, wo_pad)

    agg2p = _sc_edge_kernel(cpad, cpw, npad, "partial")(h2p, src_slab, dst_slab)

    out = pl.pallas_call(
        functools.partial(_tc_head, cpad=cpad, nclass=n_class),
        grid=grid,
        in_specs=[_row_spec(cpad), _row_spec(cpad), _row_spec(cpad),
                  _row_spec(LANES), _row_spec(LANES), _full_spec((1, n_hid)),
                  _full_spec((n_hid, cpad)), _full_spec((1, cpad))],
        out_specs=_row_spec(cpad),
        out_shape=jax.ShapeDtypeStruct((npad, cpad), jnp.float32),
    )(agg2p[0], agg2p[1], h2p, d0, d1, b2_2d, wo_pad, bo_pad)

    return out[:n_nodes, :n_class]


# speedup vs baseline: 2.0855x; 1.2737x over previous
import functools

import jax
import jax.numpy as jnp
from jax import lax
from jax.experimental import pallas as pl
from jax.experimental.pallas import tpu as pltpu
from jax.experimental.pallas import tpu_sc as plsc

NC = 2
NS = 16
CHUNK = 128
EXPORT_CHUNK = 128
LANES = 16


def _sc_edge_kernel(d_feat, cpw, npad, mode, spmem_table=False):
    rows_pt = npad // NS
    cpw_eff = 2 * cpw if mode == "split" else cpw
    mesh = plsc.VectorSubcoreMesh(core_axis_name="c", subcore_axis_name="s")

    scratch = [
        pltpu.VMEM((cpw_eff, CHUNK), jnp.int32),
        pltpu.VMEM((4, CHUNK, d_feat), jnp.float32),
        pltpu.VMEM((EXPORT_CHUNK, d_feat), jnp.float32),
        pltpu.VMEM_SHARED((npad, d_feat), jnp.float32),
        [pltpu.SemaphoreType.DMA] * 4,
    ]
    if mode != "deg":
        scratch = [pltpu.VMEM((cpw_eff, CHUNK), jnp.int32)] + scratch
        if spmem_table:
            scratch = scratch + [
                pltpu.VMEM_SHARED((npad, d_feat), jnp.float32)]

    def body(*refs):
        table_sh = None
        if mode == "deg":
            (dst_hbm, out_hbm, dst_v, rows_v, stage_v, acc_sh, sem) = refs
        elif spmem_table:
            (table_hbm, src_hbm, dst_hbm, out_hbm,
             src_v, dst_v, rows_v, stage_v, acc_sh, sem, table_sh) = refs
        else:
            (table_hbm, src_hbm, dst_hbm, out_hbm,
             src_v, dst_v, rows_v, stage_v, acc_sh, sem) = refs
        cid = lax.axis_index("c")
        sid = lax.axis_index("s")

        zero16 = jnp.zeros((LANES,), jnp.float32)

        def zero_stage(i, carry):
            for j in range(d_feat // LANES):
                stage_v[i, pl.ds(j * LANES, LANES)] = zero16
            return carry
        lax.fori_loop(0, EXPORT_CHUNK, zero_stage, 0)

        if mode == "deg":
            one16 = jnp.full((LANES,), 1.0, jnp.float32)

            def fill_ones(i, carry):
                for j in range(d_feat // LANES):
                    rows_v[0, i, pl.ds(j * LANES, LANES)] = one16
                return carry
            lax.fori_loop(0, CHUNK, fill_ones, 0)

        def zero_acc(i, carry):
            r0 = sid * rows_pt + i * EXPORT_CHUNK
            pltpu.sync_copy(stage_v, acc_sh.at[pl.ds(r0, EXPORT_CHUNK)])
            return carry
        lax.fori_loop(0, rows_pt // EXPORT_CHUNK, zero_acc, 0)

        if table_sh is not None:
            def load_table(i, carry):
                r0 = sid * rows_pt + i * EXPORT_CHUNK
                h0 = cid * npad + r0 if mode == "split" else r0
                pltpu.sync_copy(table_hbm.at[pl.ds(h0, EXPORT_CHUNK)], stage_v)
                pltpu.sync_copy(stage_v, table_sh.at[pl.ds(r0, EXPORT_CHUNK)])
                return carry
            lax.fori_loop(0, rows_pt // EXPORT_CHUNK, load_table, 0)

        src_plane = 0 if spmem_table else cid
        if mode == "split":
            pltpu.sync_copy(dst_hbm.at[sid], dst_v)
            pltpu.sync_copy(src_hbm.at[src_plane, sid], src_v)
        else:
            col = cid * cpw
            pltpu.sync_copy(dst_hbm.at[sid, pl.ds(col, cpw)], dst_v)
            if mode == "partial":
                pltpu.sync_copy(src_hbm.at[0, sid, pl.ds(col, cpw)], src_v)

        plsc.subcore_barrier()

        if mode == "deg":
            def edge_body(j, carry):
                pltpu.sync_copy(rows_v.at[0], acc_sh.at[dst_v.at[j]], add=True)
                return carry
            lax.fori_loop(0, cpw_eff, edge_body, 0)
        else:
            table = table_sh if table_sh is not None else table_hbm

            def edge_group(g, carry):
                descs = []
                for b in range(4):
                    j = 4 * g + b
                    descs.append(pltpu.async_copy(
                        table.at[src_v.at[j]], rows_v.at[b], sem[b]))
                for b in range(4):
                    j = 4 * g + b
                    descs[b].wait()
                    pltpu.sync_copy(rows_v.at[b], acc_sh.at[dst_v.at[j]],
                                    add=True)
                return carry
            lax.fori_loop(0, cpw_eff // 4, edge_group, 0)

        plsc.subcore_barrier()

        def export(i, carry):
            r0 = sid * rows_pt + i * EXPORT_CHUNK
            pltpu.sync_copy(acc_sh.at[pl.ds(r0, EXPORT_CHUNK)], stage_v)
            pltpu.sync_copy(stage_v, out_hbm.at[cid, pl.ds(r0, EXPORT_CHUNK)])
            return carry
        lax.fori_loop(0, rows_pt // EXPORT_CHUNK, export, 0)

    return functools.partial(
        pl.kernel,
        out_type=jax.ShapeDtypeStruct((NC, npad, d_feat), jnp.float32),
        mesh=mesh,
        scratch_types=scratch,
        compiler_params=pltpu.CompilerParams(use_tc_tiling_on_sc=False),
    )(body)


BLK = 256
_DOT = dict(preferred_element_type=jnp.float32, precision=lax.Precision.HIGHEST)


def _tc_scale_matmul(x_ref, w1_ref, d0_ref, d1_ref, out_ref):
    dis = lax.rsqrt(d0_ref[:, :1] + d1_ref[:, :1] + 1.0)
    h = jnp.dot(x_ref[:, :], w1_ref[:, :], **_DOT) * dis
    half = h.shape[1] // 2
    out_ref[0, :, :] = h[:, :half]
    out_ref[1, :, :] = h[:, half:]


def _tc_mid(ha_ref, hb_ref, ga_ref, gb_ref,
            d0_ref, d1_ref, b1_ref, w2_ref, wo_ref, out_ref):
    dis = lax.rsqrt(d0_ref[:, :1] + d1_ref[:, :1] + 1.0)
    agg = jnp.concatenate([ga_ref[:, :] + ha_ref[:, :],
                           gb_ref[:, :] + hb_ref[:, :]], axis=1)
    z = jnp.maximum(dis * agg + b1_ref[:, :], 0.0)
    h2 = jnp.dot(jnp.dot(z, w2_ref[:, :], **_DOT), wo_ref[:, :], **_DOT)
    out_ref[:, :] = h2 * dis


def _tc_head(q0_ref, q1_ref, h2p_ref, d0_ref, d1_ref, b2_ref, wo_ref, bo_ref,
             out_ref, *, cpad, nclass):
    dis = lax.rsqrt(d0_ref[:, :1] + d1_ref[:, :1] + 1.0)
    bias = jnp.dot(b2_ref[:, :], wo_ref[:, :], **_DOT) + bo_ref[:, :]
    u = dis * (q0_ref[:, :] + q1_ref[:, :] + h2p_ref[:, :]) + bias
    col = lax.broadcasted_iota(jnp.int32, (BLK, cpad), 1)
    valid = col < nclass
    um = jnp.where(valid, u, jnp.float32(-1e30))
    m = jnp.max(um, axis=1, keepdims=True)
    e = jnp.where(valid, jnp.exp(um - m), 0.0)
    s = jnp.sum(e, axis=1, keepdims=True)
    out_ref[:, :] = u - m - jnp.log(s)


def _row_spec(width):
    return pl.BlockSpec((BLK, width), lambda i: (i, 0))


def _full_spec(shape):
    return pl.BlockSpec(shape, lambda i: (0,) * len(shape))


def kernel(x, edge_index, W1, b1, W2, b2, Wo, bo):
    n_nodes, d_feat = x.shape
    n_hid = W1.shape[1]
    half = n_hid // 2
    n_class = Wo.shape[1]
    cpad = ((n_class + LANES - 1) // LANES) * LANES
    align = NS * EXPORT_CHUNK
    npad = ((n_nodes + 1 + align - 1) // align) * align
    n_edges = edge_index.shape[1]
    cpw = (n_edges + NC * NS * CHUNK - 1) // (NC * NS * CHUNK)
    cpw = ((cpw + 3) // 4) * 4
    e_pad = NC * NS * CHUNK * cpw

    src = edge_index[0].astype(jnp.int32)
    dst = edge_index[1].astype(jnp.int32)
    spare = npad - n_nodes
    fill = n_nodes + jnp.arange(e_pad - n_edges, dtype=jnp.int32) % spare
    src_flat = jnp.concatenate([src, fill]).reshape(NS, NC * cpw, CHUNK)
    dst_slab = jnp.concatenate([dst, fill]).reshape(NS, NC * cpw, CHUNK)
    src_slab = jnp.stack([src_flat, src_flat + npad])

    x_pad = jnp.pad(x, ((0, npad - n_nodes), (0, 0)))
    wo_pad = jnp.pad(Wo, ((0, 0), (0, cpad - n_class)))
    bo_pad = jnp.pad(bo, (0, cpad - n_class)).reshape(1, cpad)
    b1_2d = b1.reshape(1, n_hid)
    b2_2d = b2.reshape(1, n_hid)

    grid = (npad // BLK,)

    degp = _sc_edge_kernel(LANES, cpw, npad, "deg")(dst_slab)
    d0, d1 = degp[0], degp[1]

    hs = pl.pallas_call(
        _tc_scale_matmul,
        grid=grid,
        in_specs=[_row_spec(d_feat), _full_spec((d_feat, n_hid)),
                  _row_spec(LANES), _row_spec(LANES)],
        out_specs=pl.BlockSpec((NC, BLK, half), lambda i: (0, i, 0)),
        out_shape=jax.ShapeDtypeStruct((NC, npad, half), jnp.float32),
    )(x_pad, W1, d0, d1)

    table = hs.reshape(NC * npad, half)
    agg = _sc_edge_kernel(half, cpw, npad, "split")(table, src_slab, dst_slab)

    h2p = pl.pallas_call(
        _tc_mid,
        grid=grid,
        in_specs=[_row_spec(half), _row_spec(half),
                  _row_spec(half), _row_spec(half),
                  _row_spec(LANES), _row_spec(LANES), _full_spec((1, n_hid)),
                  _full_spec((n_hid, n_hid)), _full_spec((n_hid, cpad))],
        out_specs=_row_spec(cpad),
        out_shape=jax.ShapeDtypeStruct((npad, cpad), jnp.float32),
    )(hs[0], hs[1], agg[0], agg[1], d0, d1, b1_2d, W2, wo_pad)

    agg2p = _sc_edge_kernel(cpad, cpw, npad, "partial", spmem_table=True)(
        h2p, src_slab, dst_slab)

    out = pl.pallas_call(
        functools.partial(_tc_head, cpad=cpad, nclass=n_class),
        grid=grid,
        in_specs=[_row_spec(cpad), _row_spec(cpad), _row_spec(cpad),
                  _row_spec(LANES), _row_spec(LANES), _full_spec((1, n_hid)),
                  _full_spec((n_hid, cpad)), _full_spec((1, cpad))],
        out_specs=_row_spec(cpad),
        out_shape=jax.ShapeDtypeStruct((npad, cpad), jnp.float32),
    )(agg2p[0], agg2p[1], h2p, d0, d1, b2_2d, wo_pad, bo_pad)

    return out[:n_nodes, :n_class]


# speedup vs baseline: 2.2250x; 1.0669x over previous
import functools

import jax
import jax.numpy as jnp
from jax import lax
from jax.experimental import pallas as pl
from jax.experimental.pallas import tpu as pltpu
from jax.experimental.pallas import tpu_sc as plsc

NC = 2
NS = 16
CHUNK = 128
EXPORT_CHUNK = 128
LANES = 16


def _sc_edge_kernel(d_feat, cpw, npad, mode, spmem_table=False):
    rows_pt = npad // NS
    cpw_eff = 2 * cpw if mode == "split" else cpw
    mesh = plsc.VectorSubcoreMesh(core_axis_name="c", subcore_axis_name="s")

    scratch = [
        pltpu.VMEM((cpw_eff, CHUNK), jnp.int32),
        pltpu.VMEM((4, CHUNK, d_feat), jnp.float32),
        pltpu.VMEM((EXPORT_CHUNK, d_feat), jnp.float32),
        pltpu.VMEM_SHARED((npad, d_feat), jnp.float32),
        [pltpu.SemaphoreType.DMA] * 4,
    ]
    if mode != "deg":
        scratch = [pltpu.VMEM((cpw_eff, CHUNK), jnp.int32)] + scratch
        if spmem_table:
            scratch = scratch + [
                pltpu.VMEM_SHARED((npad, d_feat), jnp.float32)]

    def body(*refs):
        table_sh = None
        if mode == "deg":
            (dst_hbm, out_hbm, dst_v, rows_v, stage_v, acc_sh, sem) = refs
        elif spmem_table:
            (table_hbm, src_hbm, dst_hbm, out_hbm,
             src_v, dst_v, rows_v, stage_v, acc_sh, sem, table_sh) = refs
        else:
            (table_hbm, src_hbm, dst_hbm, out_hbm,
             src_v, dst_v, rows_v, stage_v, acc_sh, sem) = refs
        cid = lax.axis_index("c")
        sid = lax.axis_index("s")

        zero16 = jnp.zeros((LANES,), jnp.float32)

        def zero_stage(i, carry):
            for j in range(d_feat // LANES):
                stage_v[i, pl.ds(j * LANES, LANES)] = zero16
            return carry
        lax.fori_loop(0, EXPORT_CHUNK, zero_stage, 0)

        if mode == "deg":
            one16 = jnp.full((LANES,), 1.0, jnp.float32)

            def fill_ones(i, carry):
                for j in range(d_feat // LANES):
                    rows_v[0, i, pl.ds(j * LANES, LANES)] = one16
                return carry
            lax.fori_loop(0, CHUNK, fill_ones, 0)

        def zero_acc(i, carry):
            r0 = sid * rows_pt + i * EXPORT_CHUNK
            pltpu.sync_copy(stage_v, acc_sh.at[pl.ds(r0, EXPORT_CHUNK)])
            return carry
        lax.fori_loop(0, rows_pt // EXPORT_CHUNK, zero_acc, 0)

        if table_sh is not None:
            def load_table(i, carry):
                r0 = sid * rows_pt + i * EXPORT_CHUNK
                h0 = cid * npad + r0 if mode == "split" else r0
                pltpu.sync_copy(table_hbm.at[pl.ds(h0, EXPORT_CHUNK)], stage_v)
                pltpu.sync_copy(stage_v, table_sh.at[pl.ds(r0, EXPORT_CHUNK)])
                return carry
            lax.fori_loop(0, rows_pt // EXPORT_CHUNK, load_table, 0)

        src_plane = 0 if spmem_table else cid
        if mode == "split":
            pltpu.sync_copy(dst_hbm.at[sid], dst_v)
            pltpu.sync_copy(src_hbm.at[src_plane, sid], src_v)
        else:
            col = cid * cpw
            pltpu.sync_copy(dst_hbm.at[sid, pl.ds(col, cpw)], dst_v)
            if mode == "partial":
                pltpu.sync_copy(src_hbm.at[0, sid, pl.ds(col, cpw)], src_v)

        plsc.subcore_barrier()

        if mode == "deg":
            def edge_body(j, carry):
                pltpu.sync_copy(rows_v.at[0], acc_sh.at[dst_v.at[j]], add=True)
                return carry
            lax.fori_loop(0, cpw_eff, edge_body, 0)
        else:
            table = table_sh if table_sh is not None else table_hbm

            def edge_group(g, carry):
                descs = []
                for b in range(4):
                    j = 4 * g + b
                    descs.append(pltpu.async_copy(
                        table.at[src_v.at[j]], rows_v.at[b], sem[b]))
                for b in range(4):
                    j = 4 * g + b
                    descs[b].wait()
                    pltpu.sync_copy(rows_v.at[b], acc_sh.at[dst_v.at[j]],
                                    add=True)
                return carry
            lax.fori_loop(0, cpw_eff // 4, edge_group, 0)

        plsc.subcore_barrier()

        def export(i, carry):
            r0 = sid * rows_pt + i * EXPORT_CHUNK
            pltpu.sync_copy(acc_sh.at[pl.ds(r0, EXPORT_CHUNK)], stage_v)
            pltpu.sync_copy(stage_v, out_hbm.at[cid, pl.ds(r0, EXPORT_CHUNK)])
            return carry
        lax.fori_loop(0, rows_pt // EXPORT_CHUNK, export, 0)

    return functools.partial(
        pl.kernel,
        out_type=jax.ShapeDtypeStruct((NC, npad, d_feat), jnp.float32),
        mesh=mesh,
        scratch_types=scratch,
        compiler_params=pltpu.CompilerParams(use_tc_tiling_on_sc=False),
    )(body)


BLK = 256
_DOT = dict(preferred_element_type=jnp.float32, precision=lax.Precision.HIGHEST)


def _tc_scale_matmul(x_ref, w1_ref, d_ref, out_ref):
    dis = lax.rsqrt(d_ref[0, :, :1] + d_ref[1, :, :1] + 1.0)
    h = jnp.dot(x_ref[:, :], w1_ref[:, :], **_DOT) * dis
    half = h.shape[1] // 2
    out_ref[0, :, :] = h[:, :half]
    out_ref[1, :, :] = h[:, half:]


def _tc_mid(h_ref, g_ref, d_ref, b1_ref, w2_ref, wo_ref, out_ref):
    dis = lax.rsqrt(d_ref[0, :, :1] + d_ref[1, :, :1] + 1.0)
    agg = jnp.concatenate([g_ref[0, :, :] + h_ref[0, :, :],
                           g_ref[1, :, :] + h_ref[1, :, :]], axis=1)
    z = jnp.maximum(dis * agg + b1_ref[:, :], 0.0)
    h2 = jnp.dot(jnp.dot(z, w2_ref[:, :], **_DOT), wo_ref[:, :], **_DOT)
    out_ref[:, :] = h2 * dis


def _tc_head(q_ref, h2p_ref, d_ref, b2_ref, wo_ref, bo_ref,
             out_ref, *, cpad, nclass):
    dis = lax.rsqrt(d_ref[0, :, :1] + d_ref[1, :, :1] + 1.0)
    bias = jnp.dot(b2_ref[:, :], wo_ref[:, :], **_DOT) + bo_ref[:, :]
    u = dis * (q_ref[0, :, :] + q_ref[1, :, :] + h2p_ref[:, :]) + bias
    col = lax.broadcasted_iota(jnp.int32, (BLK, cpad), 1)
    valid = col < nclass
    um = jnp.where(valid, u, jnp.float32(-1e30))
    m = jnp.max(um, axis=1, keepdims=True)
    e = jnp.where(valid, jnp.exp(um - m), 0.0)
    s = jnp.sum(e, axis=1, keepdims=True)
    out_ref[:, :] = u - m - jnp.log(s)


def _row_spec(width):
    return pl.BlockSpec((BLK, width), lambda i: (i, 0))


def _stack_spec(width):
    return pl.BlockSpec((NC, BLK, width), lambda i: (0, i, 0))


def _full_spec(shape):
    return pl.BlockSpec(shape, lambda i: (0,) * len(shape))


def kernel(x, edge_index, W1, b1, W2, b2, Wo, bo):
    n_nodes, d_feat = x.shape
    n_hid = W1.shape[1]
    half = n_hid // 2
    n_class = Wo.shape[1]
    cpad = ((n_class + LANES - 1) // LANES) * LANES
    align = NS * EXPORT_CHUNK
    npad = ((n_nodes + 1 + align - 1) // align) * align
    n_edges = edge_index.shape[1]
    cpw = (n_edges + NC * NS * CHUNK - 1) // (NC * NS * CHUNK)
    cpw = ((cpw + 3) // 4) * 4
    e_pad = NC * NS * CHUNK * cpw

    src = edge_index[0].astype(jnp.int32)
    dst = edge_index[1].astype(jnp.int32)
    spare = npad - n_nodes
    fill = n_nodes + jnp.arange(e_pad - n_edges, dtype=jnp.int32) % spare
    src_flat = jnp.concatenate([src, fill]).reshape(NS, NC * cpw, CHUNK)
    dst_slab = jnp.concatenate([dst, fill]).reshape(NS, NC * cpw, CHUNK)
    src_slab = jnp.stack([src_flat, src_flat + npad])

    x_pad = jnp.pad(x, ((0, npad - n_nodes), (0, 0)))
    wo_pad = jnp.pad(Wo, ((0, 0), (0, cpad - n_class)))
    bo_pad = jnp.pad(bo, (0, cpad - n_class)).reshape(1, cpad)
    b1_2d = b1.reshape(1, n_hid)
    b2_2d = b2.reshape(1, n_hid)

    grid = (npad // BLK,)

    degp = _sc_edge_kernel(LANES, cpw, npad, "deg")(dst_slab)

    hs = pl.pallas_call(
        _tc_scale_matmul,
        grid=grid,
        in_specs=[_row_spec(d_feat), _full_spec((d_feat, n_hid)),
                  _stack_spec(LANES)],
        out_specs=_stack_spec(half),
        out_shape=jax.ShapeDtypeStruct((NC, npad, half), jnp.float32),
    )(x_pad, W1, degp)

    table = hs.reshape(NC * npad, half)
    agg = _sc_edge_kernel(half, cpw, npad, "split")(table, src_slab, dst_slab)

    h2p = pl.pallas_call(
        _tc_mid,
        grid=grid,
        in_specs=[_stack_spec(half), _stack_spec(half),
                  _stack_spec(LANES), _full_spec((1, n_hid)),
                  _full_spec((n_hid, n_hid)), _full_spec((n_hid, cpad))],
        out_specs=_row_spec(cpad),
        out_shape=jax.ShapeDtypeStruct((npad, cpad), jnp.float32),
    )(hs, agg, degp, b1_2d, W2, wo_pad)

    agg2p = _sc_edge_kernel(cpad, cpw, npad, "partial", spmem_table=True)(
        h2p, src_slab, dst_slab)

    out = pl.pallas_call(
        functools.partial(_tc_head, cpad=cpad, nclass=n_class),
        grid=grid,
        in_specs=[_stack_spec(cpad), _row_spec(cpad),
                  _stack_spec(LANES), _full_spec((1, n_hid)),
                  _full_spec((n_hid, cpad)), _full_spec((1, cpad))],
        out_specs=_row_spec(cpad),
        out_shape=jax.ShapeDtypeStruct((npad, cpad), jnp.float32),
    )(agg2p, h2p, degp, b2_2d, wo_pad, bo_pad)

    return out[:n_nodes, :n_class]


# speedup vs baseline: 2.4455x; 1.0991x over previous
import functools

import jax
import jax.numpy as jnp
from jax import lax
from jax.experimental import pallas as pl
from jax.experimental.pallas import tpu as pltpu
from jax.experimental.pallas import tpu_sc as plsc

NC = 2
NS = 16
CHUNK = 128
EXPORT_CHUNK = 128
LANES = 16


def _sc_edge_kernel(d_feat, cpw, npad, mode, spmem_table=False):
    rows_pt = npad // NS
    cpw_eff = 2 * cpw if mode == "split" else cpw
    mesh = plsc.VectorSubcoreMesh(core_axis_name="c", subcore_axis_name="s")

    scratch = [
        pltpu.VMEM((cpw_eff, CHUNK), jnp.int32),
        pltpu.VMEM((4, CHUNK, d_feat), jnp.float32),
        pltpu.VMEM((EXPORT_CHUNK, d_feat), jnp.float32),
        pltpu.VMEM_SHARED((npad, d_feat), jnp.float32),
        [pltpu.SemaphoreType.DMA] * 4,
    ]
    if mode != "deg":
        scratch = [pltpu.VMEM((cpw_eff, CHUNK), jnp.int32)] + scratch
        if spmem_table:
            scratch = scratch + [
                pltpu.VMEM_SHARED((npad, d_feat), jnp.float32)]

    def body(*refs):
        table_sh = None
        if mode == "deg":
            (dst_hbm, out_hbm, dst_v, rows_v, stage_v, acc_sh, sem) = refs
        elif spmem_table:
            (table_hbm, src_hbm, dst_hbm, out_hbm,
             src_v, dst_v, rows_v, stage_v, acc_sh, sem, table_sh) = refs
        else:
            (table_hbm, src_hbm, dst_hbm, out_hbm,
             src_v, dst_v, rows_v, stage_v, acc_sh, sem) = refs
        cid = lax.axis_index("c")
        sid = lax.axis_index("s")

        zero16 = jnp.zeros((LANES,), jnp.float32)

        def zero_stage(i, carry):
            for j in range(d_feat // LANES):
                stage_v[i, pl.ds(j * LANES, LANES)] = zero16
            return carry
        lax.fori_loop(0, EXPORT_CHUNK, zero_stage, 0)

        if mode == "deg":
            one16 = jnp.full((LANES,), 1.0, jnp.float32)

            def fill_ones(i, carry):
                for j in range(d_feat // LANES):
                    rows_v[0, i, pl.ds(j * LANES, LANES)] = one16
                return carry
            lax.fori_loop(0, CHUNK, fill_ones, 0)

        def zero_acc(i, carry):
            r0 = sid * rows_pt + i * EXPORT_CHUNK
            pltpu.sync_copy(stage_v, acc_sh.at[pl.ds(r0, EXPORT_CHUNK)])
            return carry
        lax.fori_loop(0, rows_pt // EXPORT_CHUNK, zero_acc, 0)

        if table_sh is not None:
            def load_table(i, carry):
                r0 = sid * rows_pt + i * EXPORT_CHUNK
                h0 = cid * npad + r0 if mode == "split" else r0
                pltpu.sync_copy(table_hbm.at[pl.ds(h0, EXPORT_CHUNK)], stage_v)
                pltpu.sync_copy(stage_v, table_sh.at[pl.ds(r0, EXPORT_CHUNK)])
                return carry
            lax.fori_loop(0, rows_pt // EXPORT_CHUNK, load_table, 0)

        src_plane = 0 if spmem_table else cid
        if mode == "split":
            pltpu.sync_copy(dst_hbm.at[sid], dst_v)
            pltpu.sync_copy(src_hbm.at[src_plane, sid], src_v)
        else:
            col = cid * cpw
            pltpu.sync_copy(dst_hbm.at[sid, pl.ds(col, cpw)], dst_v)
            if mode == "partial":
                pltpu.sync_copy(src_hbm.at[0, sid, pl.ds(col, cpw)], src_v)

        plsc.subcore_barrier()

        if mode == "deg":
            def edge_body(j, carry):
                pltpu.sync_copy(rows_v.at[0], acc_sh.at[dst_v.at[j]], add=True)
                return carry
            lax.fori_loop(0, cpw_eff, edge_body, 0)
        else:
            table = table_sh if table_sh is not None else table_hbm

            def edge_group(g, carry):
                descs = []
                for b in range(4):
                    j = 4 * g + b
                    descs.append(pltpu.async_copy(
                        table.at[src_v.at[j]], rows_v.at[b], sem[b]))
                for b in range(4):
                    j = 4 * g + b
                    descs[b].wait()
                    pltpu.sync_copy(rows_v.at[b], acc_sh.at[dst_v.at[j]],
                                    add=True)
                return carry
            lax.fori_loop(0, cpw_eff // 4, edge_group, 0)

        plsc.subcore_barrier()

        def export(i, carry):
            r0 = sid * rows_pt + i * EXPORT_CHUNK
            pltpu.sync_copy(acc_sh.at[pl.ds(r0, EXPORT_CHUNK)], stage_v)
            pltpu.sync_copy(stage_v, out_hbm.at[cid, pl.ds(r0, EXPORT_CHUNK)])
            return carry
        lax.fori_loop(0, rows_pt // EXPORT_CHUNK, export, 0)

    return functools.partial(
        pl.kernel,
        out_type=jax.ShapeDtypeStruct((NC, npad, d_feat), jnp.float32),
        mesh=mesh,
        scratch_types=scratch,
        compiler_params=pltpu.CompilerParams(use_tc_tiling_on_sc=False),
    )(body)


BLK = 512
_DOT = dict(preferred_element_type=jnp.float32, precision=lax.Precision.HIGHEST)


def _tc_scale_matmul(x_ref, w1_ref, d_ref, out_ref):
    dis = lax.rsqrt(d_ref[0, :, :1] + d_ref[1, :, :1] + 1.0)
    h = jnp.dot(x_ref[:, :], w1_ref[:, :], **_DOT) * dis
    half = h.shape[1] // 2
    out_ref[0, :, :] = h[:, :half]
    out_ref[1, :, :] = h[:, half:]


def _tc_mid(h_ref, g_ref, d_ref, b1_ref, w2_ref, wo_ref, out_ref):
    dis = lax.rsqrt(d_ref[0, :, :1] + d_ref[1, :, :1] + 1.0)
    agg = jnp.concatenate([g_ref[0, :, :] + h_ref[0, :, :],
                           g_ref[1, :, :] + h_ref[1, :, :]], axis=1)
    z = jnp.maximum(dis * agg + b1_ref[:, :], 0.0)
    h2 = jnp.dot(jnp.dot(z, w2_ref[:, :], **_DOT), wo_ref[:, :], **_DOT)
    out_ref[:, :] = h2 * dis


def _tc_head(q_ref, h2p_ref, d_ref, b2_ref, wo_ref, bo_ref,
             out_ref, *, cpad, nclass):
    dis = lax.rsqrt(d_ref[0, :, :1] + d_ref[1, :, :1] + 1.0)
    bias = jnp.dot(b2_ref[:, :], wo_ref[:, :], **_DOT) + bo_ref[:, :]
    u = dis * (q_ref[0, :, :] + q_ref[1, :, :] + h2p_ref[:, :]) + bias
    col = lax.broadcasted_iota(jnp.int32, (BLK, cpad), 1)
    valid = col < nclass
    um = jnp.where(valid, u, jnp.float32(-1e30))
    m = jnp.max(um, axis=1, keepdims=True)
    e = jnp.where(valid, jnp.exp(um - m), 0.0)
    s = jnp.sum(e, axis=1, keepdims=True)
    out_ref[:, :] = u - m - jnp.log(s)


def _row_spec(width):
    return pl.BlockSpec((BLK, width), lambda i: (i, 0))


def _stack_spec(width):
    return pl.BlockSpec((NC, BLK, width), lambda i: (0, i, 0))


def _full_spec(shape):
    return pl.BlockSpec(shape, lambda i: (0,) * len(shape))


def kernel(x, edge_index, W1, b1, W2, b2, Wo, bo):
    n_nodes, d_feat = x.shape
    n_hid = W1.shape[1]
    half = n_hid // 2
    n_class = Wo.shape[1]
    cpad = ((n_class + LANES - 1) // LANES) * LANES
    align = NS * EXPORT_CHUNK
    npad = ((n_nodes + 1 + align - 1) // align) * align
    n_edges = edge_index.shape[1]
    cpw = (n_edges + NC * NS * CHUNK - 1) // (NC * NS * CHUNK)
    cpw = ((cpw + 3) // 4) * 4
    e_pad = NC * NS * CHUNK * cpw

    src = edge_index[0].astype(jnp.int32)
    dst = edge_index[1].astype(jnp.int32)
    spare = npad - n_nodes
    fill = n_nodes + jnp.arange(e_pad - n_edges, dtype=jnp.int32) % spare
    src_flat = jnp.concatenate([src, fill]).reshape(NS, NC * cpw, CHUNK)
    dst_slab = jnp.concatenate([dst, fill]).reshape(NS, NC * cpw, CHUNK)
    src_slab = jnp.stack([src_flat, src_flat + npad])

    x_pad = jnp.pad(x, ((0, npad - n_nodes), (0, 0)))
    wo_pad = jnp.pad(Wo, ((0, 0), (0, cpad - n_class)))
    bo_pad = jnp.pad(bo, (0, cpad - n_class)).reshape(1, cpad)
    b1_2d = b1.reshape(1, n_hid)
    b2_2d = b2.reshape(1, n_hid)

    grid = (npad // BLK,)

    degp = _sc_edge_kernel(LANES, cpw, npad, "deg")(dst_slab)

    hs = pl.pallas_call(
        _tc_scale_matmul,
        grid=grid,
        in_specs=[_row_spec(d_feat), _full_spec((d_feat, n_hid)),
                  _stack_spec(LANES)],
        out_specs=_stack_spec(half),
        out_shape=jax.ShapeDtypeStruct((NC, npad, half), jnp.float32),
    )(x_pad, W1, degp)

    table = hs.reshape(NC * npad, half)
    agg = _sc_edge_kernel(half, cpw, npad, "split")(table, src_slab, dst_slab)

    h2p = pl.pallas_call(
        _tc_mid,
        grid=grid,
        in_specs=[_stack_spec(half), _stack_spec(half),
                  _stack_spec(LANES), _full_spec((1, n_hid)),
                  _full_spec((n_hid, n_hid)), _full_spec((n_hid, cpad))],
        out_specs=_row_spec(cpad),
        out_shape=jax.ShapeDtypeStruct((npad, cpad), jnp.float32),
    )(hs, agg, degp, b1_2d, W2, wo_pad)

    agg2p = _sc_edge_kernel(cpad, cpw, npad, "partial", spmem_table=True)(
        h2p, src_slab, dst_slab)

    out = pl.pallas_call(
        functools.partial(_tc_head, cpad=cpad, nclass=n_class),
        grid=grid,
        in_specs=[_stack_spec(cpad), _row_spec(cpad),
                  _stack_spec(LANES), _full_spec((1, n_hid)),
                  _full_spec((n_hid, cpad)), _full_spec((1, cpad))],
        out_specs=_row_spec(cpad),
        out_shape=jax.ShapeDtypeStruct((npad, cpad), jnp.float32),
    )(agg2p, h2p, degp, b2_2d, wo_pad, bo_pad)

    return out[:n_nodes, :n_class]
